# Initial kernel scaffold; baseline (speedup 1.0000x reference)
import jax
import jax.numpy as jnp
from jax.experimental import pallas as pl


def kernel(bbox_pred, conf_pred, anchors, gt_boxes):
    raise NotImplementedError("write your pallas kernel here")


# speedup vs baseline: 6.4300x; 6.4300x over previous
import jax
import jax.numpy as jnp
from jax import lax
from jax.experimental import pallas as pl
from jax.experimental.pallas import tpu as pltpu

_B, _A, _G = 8, 16384, 24
_R = 128
_C = 128
_IOU_TH, _NEG_POS_RATIO, _CONF_W, _LOC_W, _BETA = 0.5, 3, 2.0, 1.0, 0.05


def _smooth_l1(d):
    ad = jnp.abs(d)
    return jnp.where(ad < _BETA, 0.5 * ad * ad / _BETA, ad - 0.5 * _BETA)


def _loss_body(px1, px2, py1, py2, conf, ax1, ax2, ay1, ay2, gt_ref, out_ref):
    iota = (lax.broadcasted_iota(jnp.int32, (_R, _C), 0) * _C
            + lax.broadcasted_iota(jnp.int32, (_R, _C), 1))
    a1 = ax1[...]
    a2 = ax2[...]
    b1 = ay1[...]
    b2 = ay2[...]
    area_a = (a2 - a1) * (b2 - b1)

    total_loc = jnp.float32(0.0)
    total_conf = jnp.float32(0.0)
    num_pos = jnp.int32(0)

    for i in range(_B):
        x1 = px1[i]
        x2 = px2[i]
        y1 = py1[i]
        y2 = py2[i]
        p = conf[i]
        bsum = jnp.sum(x1) + jnp.sum(x2) + jnp.sum(y1) + jnp.sum(y2)
        skip = (bsum == 0.0) | (jnp.max(p) < 0.01)

        def g_body(g, carry):
            best, mcx, mcy, mw, mh, force = carry
            gx1 = gt_ref[i, g, 0]
            gy1 = gt_ref[i, g, 1]
            gx2 = gt_ref[i, g, 2]
            gy2 = gt_ref[i, g, 3]
            whx = jnp.clip(jnp.minimum(a2, gx2) - jnp.maximum(a1, gx1), 0.0, None)
            why = jnp.clip(jnp.minimum(b2, gy2) - jnp.maximum(b1, gy1), 0.0, None)
            inter = whx * why
            area_b = (gx2 - gx1) * (gy2 - gy1)
            union = area_a + area_b - inter
            iou = inter / jnp.maximum(union, 1e-9)
            upd = iou > best
            best = jnp.where(upd, iou, best)
            mcx = jnp.where(upd, (gx1 + gx2) * 0.5, mcx)
            mcy = jnp.where(upd, (gy1 + gy2) * 0.5, mcy)
            mw = jnp.where(upd, gx2 - gx1, mw)
            mh = jnp.where(upd, gy2 - gy1, mh)
            m = jnp.max(iou)
            aidx = jnp.min(jnp.where(iou == m, iota, jnp.int32(_A)))
            force = jnp.maximum(force, jnp.where(iota == aidx, 1.0, 0.0))
            return best, mcx, mcy, mw, mh, force

        zero = jnp.zeros((_R, _C), jnp.float32)
        best, mcx, mcy, mw, mh, force = lax.fori_loop(
            0, _G, g_body,
            (jnp.full((_R, _C), -1.0, jnp.float32), zero, zero, zero, zero,
             zero))

        pos = (best > _IOU_TH) | (force > 0.0)
        posf = pos.astype(jnp.float32)
        np_f = jnp.sum(posf)
        np_i = np_f.astype(jnp.int32)

        ll = (_smooth_l1((x1 + x2) * 0.5 - mcx)
              + _smooth_l1((y1 + y2) * 0.5 - mcy)
              + _smooth_l1((x2 - x1) - mw)
              + _smooth_l1((y2 - y1) - mh))
        loc_i = jnp.sum(ll * posf)

        logp = jnp.maximum(jnp.log(p), -100.0)
        log1mp = jnp.maximum(jnp.log(1.0 - p), -100.0)
        pos_conf = jnp.sum(posf * (-logp))
        neg = jnp.abs(jnp.where(pos, 0.0, -log1mp))
        k = jnp.minimum(np_i * _NEG_POS_RATIO, _A - np_i)
        kf = k.astype(jnp.float32)

        nbits = lax.bitcast_convert_type(neg, jnp.int32)

        def bit_body(j, t):
            cand = t | (jnp.int32(1) << (30 - j))
            cnt = jnp.sum((nbits >= cand).astype(jnp.int32))
            return jnp.where(cnt >= k, cand, t)

        t_bits = lax.fori_loop(0, 31, bit_body, jnp.int32(0))
        t_val = lax.bitcast_convert_type(t_bits, jnp.float32)
        gt_mask = nbits > t_bits
        cnt_gt = jnp.sum(gt_mask.astype(jnp.float32))
        sum_gt = jnp.sum(jnp.where(gt_mask, neg, 0.0))
        neg_conf = jnp.where(k > 0, sum_gt + (kf - cnt_gt) * t_val, 0.0)

        conf_i = (1.5 * pos_conf / jnp.maximum(np_f, 1.0)
                  + neg_conf / jnp.maximum(kf, 1.0))
        total_conf = total_conf + jnp.where(skip, jnp.float32(5.0), conf_i)
        total_loc = total_loc + jnp.where(skip, jnp.float32(0.0), loc_i)
        num_pos = num_pos + jnp.where(skip, jnp.int32(0), np_i)

    num_pos = jnp.maximum(1, num_pos)
    out = (total_loc / num_pos.astype(jnp.float32) * _LOC_W
           + total_conf / _B * _CONF_W)
    out_ref[0, 0] = out


def kernel(bbox_pred, conf_pred, anchors, gt_boxes):
    px1 = bbox_pred[:, :, 0].reshape(_B, _R, _C)
    py1 = bbox_pred[:, :, 1].reshape(_B, _R, _C)
    px2 = bbox_pred[:, :, 2].reshape(_B, _R, _C)
    py2 = bbox_pred[:, :, 3].reshape(_B, _R, _C)
    conf = conf_pred.reshape(_B, _R, _C)
    ax1 = anchors[:, 0].reshape(_R, _C)
    ay1 = anchors[:, 1].reshape(_R, _C)
    ax2 = anchors[:, 2].reshape(_R, _C)
    ay2 = anchors[:, 3].reshape(_R, _C)

    out = pl.pallas_call(
        _loss_body,
        out_shape=jax.ShapeDtypeStruct((1, 1), jnp.float32),
        in_specs=[pl.BlockSpec(memory_space=pltpu.VMEM)] * 9
        + [pl.BlockSpec(memory_space=pltpu.SMEM)],
        out_specs=pl.BlockSpec(memory_space=pltpu.SMEM),
    )(px1, px2, py1, py2, conf, ax1, ax2, ay1, ay2, gt_boxes)
    return out[0, 0]
